# Initial kernel scaffold; baseline (speedup 1.0000x reference)
import jax
import jax.numpy as jnp
from jax.experimental import pallas as pl


def kernel(source_nodes, destination_nodes, edge_times, edge_idxs, neighbor_nodes, neighbor_edge_idxs, neighbor_times, node_features, edge_features, time_w, time_b, Wq, Wk, Wv, fc1_w, fc1_b, fc2_w, fc2_b):
    raise NotImplementedError("write your pallas kernel here")


# speedup vs baseline: 4.5861x; 4.5861x over previous
import functools

import jax
import jax.numpy as jnp
from jax import lax
from jax.experimental import pallas as pl
from jax.experimental.pallas import tpu as pltpu
from jax.experimental.pallas import tpu_sc as plsc

_NC = 2
_NS = 16
_NW = _NC * _NS


def _proj_body(nf_ref, wkv_ref, wqf_ref, pkv_ref, pqf_ref):
    nf = nf_ref[...]
    pkv_ref[...] = jax.lax.dot_general(
        nf, wkv_ref[...], (((1,), (0,)), ((), ())),
        preferred_element_type=jnp.float32)
    pqf_ref[...] = jax.lax.dot_general(
        nf, wqf_ref[...], (((1,), (0,)), ((), ())),
        preferred_element_type=jnp.float32)


def _gather_body(C, SPW, K, nodes_hbm, misc_hbm, pkv_hbm, pqf_hbm, ef_hbm,
                 kv_out, e_out, t_out, qf_out,
                 idx_v, misc_v, qf_v, kv_v, e_v, sem1, sem2):
    wid = lax.axis_index("s") * _NC + lax.axis_index("c")
    base = wid * SPW

    def chunk(g, _):
        s0 = base + g * C
        pltpu.sync_copy(nodes_hbm.at[pl.ds(s0, C)], idx_v)
        d1 = pltpu.async_copy(misc_hbm.at[idx_v], misc_v, sem1)
        d2 = pltpu.async_copy(pqf_hbm.at[idx_v], qf_v, sem1)
        d1.wait()
        d2.wait()
        ds = []
        for i in range(C):
            ds.append(pltpu.async_copy(
                pkv_hbm.at[misc_v.at[i, pl.ds(0, 2 * K)]], kv_v.at[i], sem2))
            ds.append(pltpu.async_copy(
                ef_hbm.at[misc_v.at[i, pl.ds(48, K)]], e_v.at[i], sem2))
        for d in ds:
            d.wait()
        pltpu.sync_copy(kv_v, kv_out.at[pl.ds(s0, C)])
        pltpu.sync_copy(e_v, e_out.at[pl.ds(s0, C)])
        pltpu.sync_copy(misc_v, t_out.at[pl.ds(s0, C)])
        pltpu.sync_copy(qf_v, qf_out.at[pl.ds(s0, C)])
        return _

    lax.fori_loop(0, SPW // C, chunk, 0)


def _attn_body(S, K, T, D_H, kv_ref, e_ref, t_ref, ts_ref, qf_ref, w_ref,
               b_ref, cwt_ref, cwe_ref, wqt_ref, f1_ref, f1b_ref, f2_ref,
               f2b_ref, out_ref):
    w = w_ref[0, :]
    b = b_ref[0, :]
    tg = t_ref[:, 80:80 + K]
    dt = ts_ref[...] - tg
    targ = dt[:, :, None] * w[None, None, :] + b[None, None, :]
    tenc = jnp.cos(targ)
    proj = jax.lax.dot_general(
        tenc.reshape(S * K, T), cwt_ref[...], (((1,), (0,)), ((), ())),
        preferred_element_type=jnp.float32)
    D_E = cwe_ref.shape[0]
    proj += jax.lax.dot_general(
        e_ref[...][:, :, :D_E].reshape(S * K, D_E), cwe_ref[...],
        (((1,), (0,)), ((), ())),
        preferred_element_type=jnp.float32)
    kv = (kv_ref[...].reshape(S, K, 2, D_H)
          + proj.reshape(S, K, 2, D_H))
    kmat = kv[:, :, 0, :]
    vmat = kv[:, :, 1, :]
    qc = jax.lax.dot_general(
        jnp.cos(b)[None, :], wqt_ref[...], (((1,), (0,)), ((), ())),
        preferred_element_type=jnp.float32)
    q = qf_ref[:, :D_H] + qc
    attn = jnp.sum(q[:, None, :] * kmat, axis=-1) * (D_H ** -0.5)
    m = jnp.max(attn, axis=-1, keepdims=True)
    p = jnp.exp(attn - m)
    a = p / jnp.sum(p, axis=-1, keepdims=True)
    agg = jnp.sum(a[:, :, None] * vmat, axis=1)
    h = jax.lax.dot_general(
        agg, f1_ref[...], (((1,), (0,)), ((), ())),
        preferred_element_type=jnp.float32)
    h = jnp.maximum(h + qf_ref[:, D_H:] + f1b_ref[0, :], 0.0)
    out = jax.lax.dot_general(
        h, f2_ref[...], (((1,), (0,)), ((), ())),
        preferred_element_type=jnp.float32)
    out_ref[...] = out + f2b_ref[0, :]


def kernel(source_nodes, destination_nodes, edge_times, edge_idxs,
           neighbor_nodes, neighbor_edge_idxs, neighbor_times,
           node_features, edge_features, time_w, time_b,
           Wq, Wk, Wv, fc1_w, fc1_b, fc2_w, fc2_b):
    del edge_idxs
    B = source_nodes.shape[0]
    N2 = 2 * B
    N, K = neighbor_nodes.shape
    D = node_features.shape[1]
    D_E = edge_features.shape[1]
    T = time_w.shape[0]
    D_H = Wq.shape[1]

    nodes = jnp.concatenate([source_nodes, destination_nodes]).astype(jnp.int32)
    ts2 = jnp.concatenate([edge_times, edge_times]).reshape(N2, 1)
    nbrn = neighbor_nodes.astype(jnp.int32)
    nbre = neighbor_edge_idxs.astype(jnp.int32)
    tbits = jax.lax.bitcast_convert_type(neighbor_times, jnp.int32)
    kvidx = jnp.stack([2 * nbrn, 2 * nbrn + 1], axis=-1).reshape(N, 2 * K)
    misc_tab = jnp.concatenate(
        [kvidx, jnp.zeros((N, 48 - 2 * K), jnp.int32),
         nbre, jnp.zeros((N, 32 - K), jnp.int32),
         tbits, jnp.zeros((N, 48 - K), jnp.int32)], axis=1)
    e_tab = jnp.pad(edge_features, ((0, 0), (0, 128 - D_E)))

    wkv = jnp.concatenate([Wk[:D], Wv[:D]], axis=1)
    wqf = jnp.concatenate([Wq[:D], fc1_w[D_H:]], axis=1)
    cwt = jnp.concatenate([Wk[D + D_E:], Wv[D + D_E:]], axis=1)
    cwe = jnp.concatenate([Wk[D:D + D_E], Wv[D:D + D_E]], axis=1)
    wqt = Wq[D:]

    pkv, pqf = pl.pallas_call(
        _proj_body,
        out_shape=(
            jax.ShapeDtypeStruct((N, 2 * D_H), jnp.float32),
            jax.ShapeDtypeStruct((N, 2 * D_H), jnp.float32),
        ),
    )(node_features, wkv, wqf)

    SPW = N2 // _NW
    C = 8
    mesh = plsc.VectorSubcoreMesh(core_axis_name="c", subcore_axis_name="s")
    gather = pl.kernel(
        functools.partial(_gather_body, C, SPW, K),
        out_type=(
            jax.ShapeDtypeStruct((N2, 2 * K, D_H), jnp.float32),
            jax.ShapeDtypeStruct((N2, K, 128), jnp.float32),
            jax.ShapeDtypeStruct((N2, 128), jnp.int32),
            jax.ShapeDtypeStruct((N2, 2 * D_H), jnp.float32),
        ),
        mesh=mesh,
        scratch_types=[
            pltpu.VMEM((C,), jnp.int32),
            pltpu.VMEM((C, 128), jnp.int32),
            pltpu.VMEM((C, 2 * D_H), jnp.float32),
            pltpu.VMEM((C, 2 * K, D_H), jnp.float32),
            pltpu.VMEM((C, K, 128), jnp.float32),
            pltpu.SemaphoreType.DMA,
            pltpu.SemaphoreType.DMA,
        ],
    )
    kv_g, e_g, t_g, qf_g = gather(nodes, misc_tab, pkv.reshape(2 * N, D_H),
                                  pqf, e_tab)
    t_g = jax.lax.bitcast_convert_type(t_g, jnp.float32)

    S = 256
    grid = (N2 // S,)
    emb = pl.pallas_call(
        functools.partial(_attn_body, S, K, T, D_H),
        grid=grid,
        in_specs=[
            pl.BlockSpec((S, 2 * K, D_H), lambda i: (i, 0, 0)),
            pl.BlockSpec((S, K, 128), lambda i: (i, 0, 0)),
            pl.BlockSpec((S, 128), lambda i: (i, 0)),
            pl.BlockSpec((S, 1), lambda i: (i, 0)),
            pl.BlockSpec((S, 2 * D_H), lambda i: (i, 0)),
            pl.BlockSpec((1, T), lambda i: (0, 0)),
            pl.BlockSpec((1, T), lambda i: (0, 0)),
            pl.BlockSpec((T, 2 * D_H), lambda i: (0, 0)),
            pl.BlockSpec((D_E, 2 * D_H), lambda i: (0, 0)),
            pl.BlockSpec((T, D_H), lambda i: (0, 0)),
            pl.BlockSpec((D_H, D), lambda i: (0, 0)),
            pl.BlockSpec((1, D), lambda i: (0, 0)),
            pl.BlockSpec((D, D), lambda i: (0, 0)),
            pl.BlockSpec((1, D), lambda i: (0, 0)),
        ],
        out_specs=pl.BlockSpec((S, D), lambda i: (i, 0)),
        out_shape=jax.ShapeDtypeStruct((N2, D), jnp.float32),
    )(kv_g, e_g, t_g, ts2, qf_g, time_w.reshape(1, T), time_b.reshape(1, T),
      cwt, cwe, wqt, fc1_w[:D_H], fc1_b.reshape(1, D), fc2_w,
      fc2_b.reshape(1, D))

    return (emb[:B], emb[B:])


# speedup vs baseline: 5.5084x; 1.2011x over previous
import functools

import jax
import jax.numpy as jnp
from jax import lax
from jax.experimental import pallas as pl
from jax.experimental.pallas import tpu as pltpu
from jax.experimental.pallas import tpu_sc as plsc

_NC = 2
_NS = 16
_NW = _NC * _NS


def _proj_body(nf_ref, wkv_ref, wqf_ref, pkv_ref, pqf_ref):
    nf = nf_ref[...]
    pkv_ref[...] = jax.lax.dot_general(
        nf, wkv_ref[...], (((1,), (0,)), ((), ())),
        preferred_element_type=jnp.float32)
    pqf_ref[...] = jax.lax.dot_general(
        nf, wqf_ref[...], (((1,), (0,)), ((), ())),
        preferred_element_type=jnp.float32)


def _gather_body(C, SPW, K, nodes_hbm, misc_hbm, pkv_hbm, pqf_hbm, ef_hbm,
                 kv_out, e_out, t_out, qf_out,
                 idx_v, misc_v, qf_v, kv_v, e_v, sem1, sem2):
    wid = lax.axis_index("s") * _NC + lax.axis_index("c")
    base = wid * SPW

    def chunk(g, _):
        s0 = base + g * C
        pltpu.sync_copy(nodes_hbm.at[pl.ds(s0, C)], idx_v)
        d1 = pltpu.async_copy(misc_hbm.at[idx_v], misc_v, sem1)
        d2 = pltpu.async_copy(pqf_hbm.at[idx_v], qf_v, sem1)
        d1.wait()
        d2.wait()
        ds = []
        for i in range(C):
            ds.append(pltpu.async_copy(
                pkv_hbm.at[misc_v.at[i, pl.ds(0, 2 * K)]], kv_v.at[i], sem2))
            ds.append(pltpu.async_copy(
                ef_hbm.at[misc_v.at[i, pl.ds(48, K)]], e_v.at[i], sem2))
        for d in ds:
            d.wait()
        pltpu.sync_copy(kv_v, kv_out.at[pl.ds(s0, C)])
        pltpu.sync_copy(e_v, e_out.at[pl.ds(s0, C)])
        pltpu.sync_copy(misc_v, t_out.at[pl.ds(s0, C)])
        pltpu.sync_copy(qf_v, qf_out.at[pl.ds(s0, C)])
        return _

    lax.fori_loop(0, SPW // C, chunk, 0)


def _attn_body(S, K, T, D_H, kv_ref, e_ref, dt_ref, qf_ref, w_ref,
               b_ref, cwt_ref, cwe_ref, wqt_ref, f1_ref, f1b_ref, f2_ref,
               f2b_ref, out_ref):
    b = b_ref[...]
    targ = dt_ref[...] * w_ref[...] + b
    tenc = jnp.cos(targ)
    proj = jax.lax.dot_general(
        tenc, cwt_ref[...], (((1,), (0,)), ((), ())),
        preferred_element_type=jnp.float32)
    D_E = cwe_ref.shape[0]
    proj += jax.lax.dot_general(
        e_ref[...][:, :, :D_E].reshape(S * K, D_E), cwe_ref[...],
        (((1,), (0,)), ((), ())),
        preferred_element_type=jnp.float32)
    kmat = kv_ref[:, :K, :] + proj[:, :D_H].reshape(S, K, D_H)
    vmat = kv_ref[:, K:, :] + proj[:, D_H:].reshape(S, K, D_H)
    qc = jax.lax.dot_general(
        jnp.cos(b), wqt_ref[...], (((1,), (0,)), ((), ())),
        preferred_element_type=jnp.float32)
    q = qf_ref[:, :D_H] + qc
    attn = jnp.sum(q[:, None, :] * kmat, axis=-1) * (D_H ** -0.5)
    m = jnp.max(attn, axis=-1, keepdims=True)
    p = jnp.exp(attn - m)
    a = p / jnp.sum(p, axis=-1, keepdims=True)
    agg = jnp.sum(a[:, :, None] * vmat, axis=1)
    h = jax.lax.dot_general(
        agg, f1_ref[...], (((1,), (0,)), ((), ())),
        preferred_element_type=jnp.float32)
    h = jnp.maximum(h + qf_ref[:, D_H:] + f1b_ref[0, :], 0.0)
    out = jax.lax.dot_general(
        h, f2_ref[...], (((1,), (0,)), ((), ())),
        preferred_element_type=jnp.float32)
    out_ref[...] = out + f2b_ref[0, :]


def kernel(source_nodes, destination_nodes, edge_times, edge_idxs,
           neighbor_nodes, neighbor_edge_idxs, neighbor_times,
           node_features, edge_features, time_w, time_b,
           Wq, Wk, Wv, fc1_w, fc1_b, fc2_w, fc2_b):
    del edge_idxs
    B = source_nodes.shape[0]
    N2 = 2 * B
    N, K = neighbor_nodes.shape
    D = node_features.shape[1]
    D_E = edge_features.shape[1]
    T = time_w.shape[0]
    D_H = Wq.shape[1]

    nodes = jnp.concatenate([source_nodes, destination_nodes]).astype(jnp.int32)
    ts2 = jnp.concatenate([edge_times, edge_times]).reshape(N2, 1)
    nbrn = neighbor_nodes.astype(jnp.int32)
    nbre = neighbor_edge_idxs.astype(jnp.int32)
    tbits = jax.lax.bitcast_convert_type(neighbor_times, jnp.int32)
    kvidx = jnp.concatenate([2 * nbrn, 2 * nbrn + 1], axis=1)
    misc_tab = jnp.concatenate(
        [kvidx, jnp.zeros((N, 48 - 2 * K), jnp.int32),
         nbre, jnp.zeros((N, 32 - K), jnp.int32),
         tbits, jnp.zeros((N, 48 - K), jnp.int32)], axis=1)
    e_tab = jnp.pad(edge_features, ((0, 0), (0, 128 - D_E)))

    wkv = jnp.concatenate([Wk[:D], Wv[:D]], axis=1)
    wqf = jnp.concatenate([Wq[:D], fc1_w[D_H:]], axis=1)
    cwt = jnp.concatenate([Wk[D + D_E:], Wv[D + D_E:]], axis=1)
    cwe = jnp.concatenate([Wk[D:D + D_E], Wv[D:D + D_E]], axis=1)
    wqt = Wq[D:]

    pkv, pqf = pl.pallas_call(
        _proj_body,
        out_shape=(
            jax.ShapeDtypeStruct((N, 2 * D_H), jnp.float32),
            jax.ShapeDtypeStruct((N, 2 * D_H), jnp.float32),
        ),
    )(node_features, wkv, wqf)

    SPW = N2 // _NW
    C = 8
    mesh = plsc.VectorSubcoreMesh(core_axis_name="c", subcore_axis_name="s")
    gather = pl.kernel(
        functools.partial(_gather_body, C, SPW, K),
        out_type=(
            jax.ShapeDtypeStruct((N2, 2 * K, D_H), jnp.float32),
            jax.ShapeDtypeStruct((N2, K, 128), jnp.float32),
            jax.ShapeDtypeStruct((N2, 128), jnp.int32),
            jax.ShapeDtypeStruct((N2, 2 * D_H), jnp.float32),
        ),
        mesh=mesh,
        scratch_types=[
            pltpu.VMEM((C,), jnp.int32),
            pltpu.VMEM((C, 128), jnp.int32),
            pltpu.VMEM((C, 2 * D_H), jnp.float32),
            pltpu.VMEM((C, 2 * K, D_H), jnp.float32),
            pltpu.VMEM((C, K, 128), jnp.float32),
            pltpu.SemaphoreType.DMA,
            pltpu.SemaphoreType.DMA,
        ],
    )
    kv_g, e_g, t_g, qf_g = gather(nodes, misc_tab, pkv.reshape(2 * N, D_H),
                                  pqf, e_tab)
    tg = jax.lax.bitcast_convert_type(t_g, jnp.float32)[:, 80:80 + K]
    dtf = (ts2 - tg).reshape(N2 * K, 1)

    S = 256
    grid = (N2 // S,)
    emb = pl.pallas_call(
        functools.partial(_attn_body, S, K, T, D_H),
        grid=grid,
        in_specs=[
            pl.BlockSpec((S, 2 * K, D_H), lambda i: (i, 0, 0)),
            pl.BlockSpec((S, K, 128), lambda i: (i, 0, 0)),
            pl.BlockSpec((S * K, 1), lambda i: (i, 0)),
            pl.BlockSpec((S, 2 * D_H), lambda i: (i, 0)),
            pl.BlockSpec((1, T), lambda i: (0, 0)),
            pl.BlockSpec((1, T), lambda i: (0, 0)),
            pl.BlockSpec((T, 2 * D_H), lambda i: (0, 0)),
            pl.BlockSpec((D_E, 2 * D_H), lambda i: (0, 0)),
            pl.BlockSpec((T, D_H), lambda i: (0, 0)),
            pl.BlockSpec((D_H, D), lambda i: (0, 0)),
            pl.BlockSpec((1, D), lambda i: (0, 0)),
            pl.BlockSpec((D, D), lambda i: (0, 0)),
            pl.BlockSpec((1, D), lambda i: (0, 0)),
        ],
        out_specs=pl.BlockSpec((S, D), lambda i: (i, 0)),
        out_shape=jax.ShapeDtypeStruct((N2, D), jnp.float32),
    )(kv_g, e_g, dtf, qf_g, time_w.reshape(1, T), time_b.reshape(1, T),
      cwt, cwe, wqt, fc1_w[:D_H], fc1_b.reshape(1, D), fc2_w,
      fc2_b.reshape(1, D))

    return (emb[:B], emb[B:])


# speedup vs baseline: 5.5163x; 1.0014x over previous
import functools

import jax
import jax.numpy as jnp
from jax import lax
from jax.experimental import pallas as pl
from jax.experimental.pallas import tpu as pltpu
from jax.experimental.pallas import tpu_sc as plsc

_NC = 2
_NS = 16
_NW = _NC * _NS


def _proj_body(nf_ref, wkv_ref, wqf_ref, pkv_ref, pqf_ref):
    nf = nf_ref[...]
    pkv_ref[...] = jax.lax.dot_general(
        nf, wkv_ref[...], (((1,), (0,)), ((), ())),
        preferred_element_type=jnp.float32)
    pqf_ref[...] = jax.lax.dot_general(
        nf, wqf_ref[...], (((1,), (0,)), ((), ())),
        preferred_element_type=jnp.float32)


def _gather_body(C, SPW, K, nodes_hbm, misc_hbm, pkv_hbm, pqf_hbm, ef_hbm,
                 kv_out, e_out, t_out, qf_out,
                 idx_v, misc_v, qf_v, kv_v, e_v, sem1, sem2):
    wid = lax.axis_index("s") * _NC + lax.axis_index("c")
    base = wid * SPW

    def chunk(g, _):
        s0 = base + g * C
        pltpu.sync_copy(nodes_hbm.at[pl.ds(s0, C)], idx_v)
        d1 = pltpu.async_copy(misc_hbm.at[idx_v], misc_v, sem1)
        d2 = pltpu.async_copy(pqf_hbm.at[idx_v], qf_v, sem1)
        d1.wait()
        d2.wait()
        ds = []
        for i in range(C):
            ds.append(pltpu.async_copy(
                pkv_hbm.at[misc_v.at[i, pl.ds(0, 2 * K)]], kv_v.at[i], sem2))
            ds.append(pltpu.async_copy(
                ef_hbm.at[misc_v.at[i, pl.ds(48, K)]], e_v.at[i], sem2))
        for d in ds:
            d.wait()
        pltpu.sync_copy(kv_v, kv_out.at[pl.ds(s0, C)])
        pltpu.sync_copy(e_v, e_out.at[pl.ds(s0, C)])
        pltpu.sync_copy(misc_v, t_out.at[pl.ds(s0, C)])
        pltpu.sync_copy(qf_v, qf_out.at[pl.ds(s0, C)])
        return _

    lax.fori_loop(0, SPW // C, chunk, 0)


def _attn_body(S, K, T, D_H, kv_ref, e_ref, dt_ref, qf_ref, w_ref,
               b_ref, cwt_ref, cwe_ref, wqt_ref, f1_ref, f1b_ref, f2_ref,
               f2b_ref, out_ref):
    b = b_ref[...]
    targ = jax.lax.dot_general(
        dt_ref[...], w_ref[...], (((1,), (0,)), ((), ())),
        preferred_element_type=jnp.float32) + b
    tenc = jnp.cos(targ)
    proj = jax.lax.dot_general(
        tenc, cwt_ref[...], (((1,), (0,)), ((), ())),
        preferred_element_type=jnp.float32)
    D_E = cwe_ref.shape[0]
    proj += jax.lax.dot_general(
        e_ref[...][:, :, :D_E].reshape(S * K, D_E), cwe_ref[...],
        (((1,), (0,)), ((), ())),
        preferred_element_type=jnp.float32)
    kmat = kv_ref[:, :K, :] + proj[:, :D_H].reshape(S, K, D_H)
    vmat = kv_ref[:, K:, :] + proj[:, D_H:].reshape(S, K, D_H)
    qc = jax.lax.dot_general(
        jnp.cos(b), wqt_ref[...], (((1,), (0,)), ((), ())),
        preferred_element_type=jnp.float32)
    q = qf_ref[:, :D_H] + qc
    attn = jnp.sum(q[:, None, :] * kmat, axis=-1) * (D_H ** -0.5)
    m = jnp.max(attn, axis=-1, keepdims=True)
    p = jnp.exp(attn - m)
    a = p / jnp.sum(p, axis=-1, keepdims=True)
    agg = jnp.sum(a[:, :, None] * vmat, axis=1)
    h = jax.lax.dot_general(
        agg, f1_ref[...], (((1,), (0,)), ((), ())),
        preferred_element_type=jnp.float32)
    h = jnp.maximum(h + qf_ref[:, D_H:] + f1b_ref[0, :], 0.0)
    out = jax.lax.dot_general(
        h, f2_ref[...], (((1,), (0,)), ((), ())),
        preferred_element_type=jnp.float32)
    out_ref[...] = out + f2b_ref[0, :]


def kernel(source_nodes, destination_nodes, edge_times, edge_idxs,
           neighbor_nodes, neighbor_edge_idxs, neighbor_times,
           node_features, edge_features, time_w, time_b,
           Wq, Wk, Wv, fc1_w, fc1_b, fc2_w, fc2_b):
    del edge_idxs
    B = source_nodes.shape[0]
    N2 = 2 * B
    N, K = neighbor_nodes.shape
    D = node_features.shape[1]
    D_E = edge_features.shape[1]
    T = time_w.shape[0]
    D_H = Wq.shape[1]

    nodes = jnp.concatenate([source_nodes, destination_nodes]).astype(jnp.int32)
    ts2 = jnp.concatenate([edge_times, edge_times]).reshape(N2, 1)
    nbrn = neighbor_nodes.astype(jnp.int32)
    nbre = neighbor_edge_idxs.astype(jnp.int32)
    tbits = jax.lax.bitcast_convert_type(neighbor_times, jnp.int32)
    kvidx = jnp.concatenate([2 * nbrn, 2 * nbrn + 1], axis=1)
    misc_tab = jnp.concatenate(
        [kvidx, jnp.zeros((N, 48 - 2 * K), jnp.int32),
         nbre, jnp.zeros((N, 32 - K), jnp.int32),
         tbits, jnp.zeros((N, 48 - K), jnp.int32)], axis=1)
    e_tab = jnp.pad(edge_features, ((0, 0), (0, 128 - D_E)))

    wkv = jnp.concatenate([Wk[:D], Wv[:D]], axis=1)
    wqf = jnp.concatenate([Wq[:D], fc1_w[D_H:]], axis=1)
    cwt = jnp.concatenate([Wk[D + D_E:], Wv[D + D_E:]], axis=1)
    cwe = jnp.concatenate([Wk[D:D + D_E], Wv[D:D + D_E]], axis=1)
    wqt = Wq[D:]

    pkv, pqf = pl.pallas_call(
        _proj_body,
        out_shape=(
            jax.ShapeDtypeStruct((N, 2 * D_H), jnp.float32),
            jax.ShapeDtypeStruct((N, 2 * D_H), jnp.float32),
        ),
    )(node_features, wkv, wqf)

    SPW = N2 // _NW
    C = 8
    mesh = plsc.VectorSubcoreMesh(core_axis_name="c", subcore_axis_name="s")
    gather = pl.kernel(
        functools.partial(_gather_body, C, SPW, K),
        out_type=(
            jax.ShapeDtypeStruct((N2, 2 * K, D_H), jnp.float32),
            jax.ShapeDtypeStruct((N2, K, 128), jnp.float32),
            jax.ShapeDtypeStruct((N2, 128), jnp.int32),
            jax.ShapeDtypeStruct((N2, 2 * D_H), jnp.float32),
        ),
        mesh=mesh,
        scratch_types=[
            pltpu.VMEM((C,), jnp.int32),
            pltpu.VMEM((C, 128), jnp.int32),
            pltpu.VMEM((C, 2 * D_H), jnp.float32),
            pltpu.VMEM((C, 2 * K, D_H), jnp.float32),
            pltpu.VMEM((C, K, 128), jnp.float32),
            pltpu.SemaphoreType.DMA,
            pltpu.SemaphoreType.DMA,
        ],
    )
    kv_g, e_g, t_g, qf_g = gather(nodes, misc_tab, pkv.reshape(2 * N, D_H),
                                  pqf, e_tab)
    tg = jax.lax.bitcast_convert_type(t_g, jnp.float32)[:, 80:80 + K]
    dtf = (ts2 - tg).reshape(N2 * K, 1)

    S = 256
    grid = (N2 // S,)
    emb = pl.pallas_call(
        functools.partial(_attn_body, S, K, T, D_H),
        grid=grid,
        in_specs=[
            pl.BlockSpec((S, 2 * K, D_H), lambda i: (i, 0, 0)),
            pl.BlockSpec((S, K, 128), lambda i: (i, 0, 0)),
            pl.BlockSpec((S * K, 1), lambda i: (i, 0)),
            pl.BlockSpec((S, 2 * D_H), lambda i: (i, 0)),
            pl.BlockSpec((1, T), lambda i: (0, 0)),
            pl.BlockSpec((1, T), lambda i: (0, 0)),
            pl.BlockSpec((T, 2 * D_H), lambda i: (0, 0)),
            pl.BlockSpec((D_E, 2 * D_H), lambda i: (0, 0)),
            pl.BlockSpec((T, D_H), lambda i: (0, 0)),
            pl.BlockSpec((D_H, D), lambda i: (0, 0)),
            pl.BlockSpec((1, D), lambda i: (0, 0)),
            pl.BlockSpec((D, D), lambda i: (0, 0)),
            pl.BlockSpec((1, D), lambda i: (0, 0)),
        ],
        out_specs=pl.BlockSpec((S, D), lambda i: (i, 0)),
        out_shape=jax.ShapeDtypeStruct((N2, D), jnp.float32),
    )(kv_g, e_g, dtf, qf_g, time_w.reshape(1, T), time_b.reshape(1, T),
      cwt, cwe, wqt, fc1_w[:D_H], fc1_b.reshape(1, D), fc2_w,
      fc2_b.reshape(1, D))

    return (emb[:B], emb[B:])


# speedup vs baseline: 7.5359x; 1.3661x over previous
import functools

import jax
import jax.numpy as jnp
from jax import lax
from jax.experimental import pallas as pl
from jax.experimental.pallas import tpu as pltpu
from jax.experimental.pallas import tpu_sc as plsc

_NC = 2
_NS = 16
_NW = _NC * _NS


def _proj_body(nf_ref, wkv_ref, wqf_ref, pkv_ref, pqf_ref):
    nf = nf_ref[...]
    pkv_ref[...] = jax.lax.dot_general(
        nf, wkv_ref[...], (((1,), (0,)), ((), ())),
        preferred_element_type=jnp.float32)
    pqf_ref[...] = jax.lax.dot_general(
        nf, wqf_ref[...], (((1,), (0,)), ((), ())),
        preferred_element_type=jnp.float32)


def _gather_body(C, SPW, K, nodes_hbm, misc_hbm, pkv_hbm, pqf_hbm, ef_hbm,
                 kv_out, e_out, t_out, qf_out,
                 idx_v, misc_v, qf_v, kv_v, e_v, sem1, sem2):
    wid = lax.axis_index("s") * _NC + lax.axis_index("c")
    base = wid * SPW

    def chunk(g, _):
        s0 = base + g * C
        pltpu.sync_copy(nodes_hbm.at[pl.ds(s0, C)], idx_v)
        d1 = pltpu.async_copy(misc_hbm.at[idx_v], misc_v, sem1)
        d2 = pltpu.async_copy(pqf_hbm.at[idx_v], qf_v, sem1)
        d1.wait()
        d2.wait()
        ds = []
        for i in range(C):
            ds.append(pltpu.async_copy(
                pkv_hbm.at[misc_v.at[i, pl.ds(0, 2 * K)]], kv_v.at[i], sem2))
            ds.append(pltpu.async_copy(
                ef_hbm.at[misc_v.at[i, pl.ds(48, K)]], e_v.at[i], sem2))
        for d in ds:
            d.wait()
        pltpu.sync_copy(kv_v, kv_out.at[pl.ds(s0, C)])
        pltpu.sync_copy(e_v, e_out.at[pl.ds(s0, C)])
        pltpu.sync_copy(misc_v, t_out.at[pl.ds(s0, C)])
        pltpu.sync_copy(qf_v, qf_out.at[pl.ds(s0, C)])
        return _

    lax.fori_loop(0, SPW // C, chunk, 0)


def _fast_cos(x):
    n = jnp.round(x * 0.15915494309189535)
    r = x - n * 6.28125
    r = r - n * 1.9353071795864769e-03
    z = r * r
    p = jnp.float32(4.779477332387385e-14)
    for c in (-1.1470745597729725e-11, 2.08767569878681e-09,
              -2.755731922398589e-07, 2.48015873015873e-05,
              -1.388888888888889e-03, 4.1666666666666664e-02,
              -0.5, 1.0):
        p = p * z + c
    return p


def _attn_body(S, K, T, D_H, kv_ref, e_ref, dt_ref, qf_ref, w_ref,
               b_ref, cwt_ref, cwe_ref, wqt_ref, f1_ref, f1b_ref, f2_ref,
               f2b_ref, out_ref):
    b = b_ref[...]
    targ = dt_ref[...] * w_ref[...] + b
    tenc = _fast_cos(targ)
    proj = jax.lax.dot_general(
        tenc, cwt_ref[...], (((1,), (0,)), ((), ())),
        preferred_element_type=jnp.float32)
    D_E = cwe_ref.shape[0]
    proj += jax.lax.dot_general(
        e_ref[...][:, :, :D_E].reshape(S * K, D_E), cwe_ref[...],
        (((1,), (0,)), ((), ())),
        preferred_element_type=jnp.float32)
    kmat = kv_ref[:, :K, :] + proj[:, :D_H].reshape(S, K, D_H)
    vmat = kv_ref[:, K:, :] + proj[:, D_H:].reshape(S, K, D_H)
    qc = jax.lax.dot_general(
        jnp.cos(b), wqt_ref[...], (((1,), (0,)), ((), ())),
        preferred_element_type=jnp.float32)
    q = qf_ref[:, :D_H] + qc
    attn = jnp.sum(q[:, None, :] * kmat, axis=-1) * (D_H ** -0.5)
    m = jnp.max(attn, axis=-1, keepdims=True)
    p = jnp.exp(attn - m)
    a = p / jnp.sum(p, axis=-1, keepdims=True)
    agg = jnp.sum(a[:, :, None] * vmat, axis=1)
    h = jax.lax.dot_general(
        agg, f1_ref[...], (((1,), (0,)), ((), ())),
        preferred_element_type=jnp.float32)
    h = jnp.maximum(h + qf_ref[:, D_H:] + f1b_ref[0, :], 0.0)
    out = jax.lax.dot_general(
        h, f2_ref[...], (((1,), (0,)), ((), ())),
        preferred_element_type=jnp.float32)
    out_ref[...] = out + f2b_ref[0, :]


def kernel(source_nodes, destination_nodes, edge_times, edge_idxs,
           neighbor_nodes, neighbor_edge_idxs, neighbor_times,
           node_features, edge_features, time_w, time_b,
           Wq, Wk, Wv, fc1_w, fc1_b, fc2_w, fc2_b):
    del edge_idxs
    B = source_nodes.shape[0]
    N2 = 2 * B
    N, K = neighbor_nodes.shape
    D = node_features.shape[1]
    D_E = edge_features.shape[1]
    T = time_w.shape[0]
    D_H = Wq.shape[1]

    nodes = jnp.concatenate([source_nodes, destination_nodes]).astype(jnp.int32)
    ts2 = jnp.concatenate([edge_times, edge_times]).reshape(N2, 1)
    nbrn = neighbor_nodes.astype(jnp.int32)
    nbre = neighbor_edge_idxs.astype(jnp.int32)
    tbits = jax.lax.bitcast_convert_type(neighbor_times, jnp.int32)
    kvidx = jnp.concatenate([2 * nbrn, 2 * nbrn + 1], axis=1)
    misc_tab = jnp.concatenate(
        [kvidx, jnp.zeros((N, 48 - 2 * K), jnp.int32),
         nbre, jnp.zeros((N, 32 - K), jnp.int32),
         tbits, jnp.zeros((N, 48 - K), jnp.int32)], axis=1)
    e_tab = jnp.pad(edge_features, ((0, 0), (0, 128 - D_E)))

    wkv = jnp.concatenate([Wk[:D], Wv[:D]], axis=1)
    wqf = jnp.concatenate([Wq[:D], fc1_w[D_H:]], axis=1)
    cwt = jnp.concatenate([Wk[D + D_E:], Wv[D + D_E:]], axis=1)
    cwe = jnp.concatenate([Wk[D:D + D_E], Wv[D:D + D_E]], axis=1)
    wqt = Wq[D:]

    pkv, pqf = pl.pallas_call(
        _proj_body,
        out_shape=(
            jax.ShapeDtypeStruct((N, 2 * D_H), jnp.float32),
            jax.ShapeDtypeStruct((N, 2 * D_H), jnp.float32),
        ),
    )(node_features, wkv, wqf)

    SPW = N2 // _NW
    C = 8
    mesh = plsc.VectorSubcoreMesh(core_axis_name="c", subcore_axis_name="s")
    gather = pl.kernel(
        functools.partial(_gather_body, C, SPW, K),
        out_type=(
            jax.ShapeDtypeStruct((N2, 2 * K, D_H), jnp.float32),
            jax.ShapeDtypeStruct((N2, K, 128), jnp.float32),
            jax.ShapeDtypeStruct((N2, 128), jnp.int32),
            jax.ShapeDtypeStruct((N2, 2 * D_H), jnp.float32),
        ),
        mesh=mesh,
        scratch_types=[
            pltpu.VMEM((C,), jnp.int32),
            pltpu.VMEM((C, 128), jnp.int32),
            pltpu.VMEM((C, 2 * D_H), jnp.float32),
            pltpu.VMEM((C, 2 * K, D_H), jnp.float32),
            pltpu.VMEM((C, K, 128), jnp.float32),
            pltpu.SemaphoreType.DMA,
            pltpu.SemaphoreType.DMA,
        ],
    )
    kv_g, e_g, t_g, qf_g = gather(nodes, misc_tab, pkv.reshape(2 * N, D_H),
                                  pqf, e_tab)
    tg = jax.lax.bitcast_convert_type(t_g, jnp.float32)[:, 80:80 + K]
    dtf = (ts2 - tg).reshape(N2 * K, 1)

    S = 256
    grid = (N2 // S,)
    emb = pl.pallas_call(
        functools.partial(_attn_body, S, K, T, D_H),
        grid=grid,
        in_specs=[
            pl.BlockSpec((S, 2 * K, D_H), lambda i: (i, 0, 0)),
            pl.BlockSpec((S, K, 128), lambda i: (i, 0, 0)),
            pl.BlockSpec((S * K, 1), lambda i: (i, 0)),
            pl.BlockSpec((S, 2 * D_H), lambda i: (i, 0)),
            pl.BlockSpec((1, T), lambda i: (0, 0)),
            pl.BlockSpec((1, T), lambda i: (0, 0)),
            pl.BlockSpec((T, 2 * D_H), lambda i: (0, 0)),
            pl.BlockSpec((D_E, 2 * D_H), lambda i: (0, 0)),
            pl.BlockSpec((T, D_H), lambda i: (0, 0)),
            pl.BlockSpec((D_H, D), lambda i: (0, 0)),
            pl.BlockSpec((1, D), lambda i: (0, 0)),
            pl.BlockSpec((D, D), lambda i: (0, 0)),
            pl.BlockSpec((1, D), lambda i: (0, 0)),
        ],
        out_specs=pl.BlockSpec((S, D), lambda i: (i, 0)),
        out_shape=jax.ShapeDtypeStruct((N2, D), jnp.float32),
    )(kv_g, e_g, dtf, qf_g, time_w.reshape(1, T), time_b.reshape(1, T),
      cwt, cwe, wqt, fc1_w[:D_H], fc1_b.reshape(1, D), fc2_w,
      fc2_b.reshape(1, D))

    return (emb[:B], emb[B:])


# speedup vs baseline: 8.4447x; 1.1206x over previous
import functools

import jax
import jax.numpy as jnp
from jax import lax
from jax.experimental import pallas as pl
from jax.experimental.pallas import tpu as pltpu
from jax.experimental.pallas import tpu_sc as plsc

_NC = 2
_NS = 16
_NW = _NC * _NS


def _proj_body(D_H, nf_ref, wkv_ref, wqf_ref, pkv_ref, pqf_ref):
    nf = nf_ref[...]
    kv = jax.lax.dot_general(
        nf, wkv_ref[...], (((1,), (0,)), ((), ())),
        preferred_element_type=jnp.float32)
    kw = jax.lax.bitcast_convert_type(
        kv[:, :D_H].astype(jnp.bfloat16), jnp.uint16).astype(jnp.uint32)
    vw = jax.lax.bitcast_convert_type(
        kv[:, D_H:].astype(jnp.bfloat16), jnp.uint16).astype(jnp.uint32)
    pkv_ref[...] = jax.lax.bitcast_convert_type(
        kw | (vw << 16), jnp.int32)
    pqf_ref[...] = jax.lax.dot_general(
        nf, wqf_ref[...], (((1,), (0,)), ((), ())),
        preferred_element_type=jnp.float32)


def _gather_body(C, SPW, K, nodes_hbm, misc_hbm, pkv_hbm, pqf_hbm, ef_hbm,
                 kv_out, e_out, t_out, qf_out,
                 idx_v, misc_v, qf_v, kv_v, e_v, sem1, sem2):
    wid = lax.axis_index("s") * _NC + lax.axis_index("c")
    base = wid * SPW

    def chunk(g, _):
        s0 = base + g * C
        pltpu.sync_copy(nodes_hbm.at[pl.ds(s0, C)], idx_v)
        d1 = pltpu.async_copy(misc_hbm.at[idx_v], misc_v, sem1)
        d2 = pltpu.async_copy(pqf_hbm.at[idx_v], qf_v, sem1)
        d1.wait()
        d2.wait()
        ds = []
        for i in range(C):
            ds.append(pltpu.async_copy(
                pkv_hbm.at[misc_v.at[i, pl.ds(0, K)]], kv_v.at[i], sem2))
            ds.append(pltpu.async_copy(
                ef_hbm.at[misc_v.at[i, pl.ds(32, K)]], e_v.at[i], sem2))
        for d in ds:
            d.wait()
        pltpu.sync_copy(kv_v, kv_out.at[pl.ds(s0, C)])
        pltpu.sync_copy(e_v, e_out.at[pl.ds(s0, C)])
        pltpu.sync_copy(misc_v, t_out.at[pl.ds(s0, C)])
        pltpu.sync_copy(qf_v, qf_out.at[pl.ds(s0, C)])
        return _

    lax.fori_loop(0, SPW // C, chunk, 0)


def _fast_cos(x):
    n = jnp.round(x * 0.15915494309189535)
    r = x - n * 6.28125
    r = r - n * 1.9353071795864769e-03
    z = r * r
    p = jnp.float32(4.779477332387385e-14)
    for c in (-1.1470745597729725e-11, 2.08767569878681e-09,
              -2.755731922398589e-07, 2.48015873015873e-05,
              -1.388888888888889e-03, 4.1666666666666664e-02,
              -0.5, 1.0):
        p = p * z + c
    return p


def _attn_body(S, K, T, D_H, kv_ref, e_ref, dt_ref, qf_ref, w_ref,
               b_ref, cwt_ref, cwe_ref, wqt_ref, f1_ref, f1b_ref, f2_ref,
               f2b_ref, out_ref):
    b = b_ref[...]
    targ = dt_ref[...] * w_ref[...] + b
    tenc = _fast_cos(targ)
    proj = jax.lax.dot_general(
        tenc, cwt_ref[...], (((1,), (0,)), ((), ())),
        preferred_element_type=jnp.float32)
    D_E = cwe_ref.shape[0]
    proj += jax.lax.dot_general(
        e_ref[...][:, :, :D_E].reshape(S * K, D_E),
        cwe_ref[...], (((1,), (0,)), ((), ())),
        preferred_element_type=jnp.float32)
    kvp = jax.lax.bitcast_convert_type(kv_ref[...], jnp.uint32)
    kf = jax.lax.bitcast_convert_type(
        (kvp & jnp.uint32(0xFFFF)).astype(jnp.uint16),
        jnp.bfloat16).astype(jnp.float32)
    vf = jax.lax.bitcast_convert_type(
        (kvp >> 16).astype(jnp.uint16), jnp.bfloat16).astype(jnp.float32)
    kmat = kf + proj[:, :D_H].reshape(S, K, D_H)
    vmat = vf + proj[:, D_H:].reshape(S, K, D_H)
    qc = jax.lax.dot_general(
        jnp.cos(b), wqt_ref[...], (((1,), (0,)), ((), ())),
        preferred_element_type=jnp.float32)
    q = qf_ref[:, :D_H] + qc
    attn = jnp.sum(q[:, None, :] * kmat, axis=-1) * (D_H ** -0.5)
    m = jnp.max(attn, axis=-1, keepdims=True)
    p = jnp.exp(attn - m)
    a = p / jnp.sum(p, axis=-1, keepdims=True)
    agg = jnp.sum(a[:, :, None] * vmat, axis=1)
    h = jax.lax.dot_general(
        agg, f1_ref[...], (((1,), (0,)), ((), ())),
        preferred_element_type=jnp.float32)
    h = jnp.maximum(h + qf_ref[:, D_H:] + f1b_ref[0, :], 0.0)
    out = jax.lax.dot_general(
        h, f2_ref[...], (((1,), (0,)), ((), ())),
        preferred_element_type=jnp.float32)
    out_ref[...] = out + f2b_ref[0, :]


def kernel(source_nodes, destination_nodes, edge_times, edge_idxs,
           neighbor_nodes, neighbor_edge_idxs, neighbor_times,
           node_features, edge_features, time_w, time_b,
           Wq, Wk, Wv, fc1_w, fc1_b, fc2_w, fc2_b):
    del edge_idxs
    B = source_nodes.shape[0]
    N2 = 2 * B
    N, K = neighbor_nodes.shape
    D = node_features.shape[1]
    D_E = edge_features.shape[1]
    T = time_w.shape[0]
    D_H = Wq.shape[1]

    nodes = jnp.concatenate([source_nodes, destination_nodes]).astype(jnp.int32)
    ts2 = jnp.concatenate([edge_times, edge_times]).reshape(N2, 1)
    nbrn = neighbor_nodes.astype(jnp.int32)
    nbre = neighbor_edge_idxs.astype(jnp.int32)
    tbits = jax.lax.bitcast_convert_type(neighbor_times, jnp.int32)
    zpad = jnp.zeros((N, 32 - K), jnp.int32)
    misc_tab = jnp.concatenate(
        [nbrn, zpad, nbre, zpad, tbits, zpad,
         jnp.zeros((N, 32), jnp.int32)], axis=1)
    e_tab = jnp.pad(edge_features, ((0, 0), (0, 128 - D_E)))

    wkv = jnp.concatenate([Wk[:D], Wv[:D]], axis=1)
    wqf = jnp.concatenate([Wq[:D], fc1_w[D_H:]], axis=1)
    cwt = jnp.concatenate([Wk[D + D_E:], Wv[D + D_E:]], axis=1)
    cwe = jnp.concatenate([Wk[D:D + D_E], Wv[D:D + D_E]], axis=1)
    wqt = Wq[D:]

    pkv, pqf = pl.pallas_call(
        functools.partial(_proj_body, D_H),
        out_shape=(
            jax.ShapeDtypeStruct((N, D_H), jnp.int32),
            jax.ShapeDtypeStruct((N, 2 * D_H), jnp.float32),
        ),
    )(node_features, wkv, wqf)

    SPW = N2 // _NW
    C = 8
    mesh = plsc.VectorSubcoreMesh(core_axis_name="c", subcore_axis_name="s")
    gather = pl.kernel(
        functools.partial(_gather_body, C, SPW, K),
        out_type=(
            jax.ShapeDtypeStruct((N2, K, 128), jnp.int32),
            jax.ShapeDtypeStruct((N2, K, 128), jnp.float32),
            jax.ShapeDtypeStruct((N2, 128), jnp.int32),
            jax.ShapeDtypeStruct((N2, 2 * D_H), jnp.float32),
        ),
        mesh=mesh,
        scratch_types=[
            pltpu.VMEM((C,), jnp.int32),
            pltpu.VMEM((C, 128), jnp.int32),
            pltpu.VMEM((C, 2 * D_H), jnp.float32),
            pltpu.VMEM((C, K, 128), jnp.int32),
            pltpu.VMEM((C, K, 128), jnp.float32),
            pltpu.SemaphoreType.DMA,
            pltpu.SemaphoreType.DMA,
        ],
    )
    kv_g, e_g, t_g, qf_g = gather(nodes, misc_tab, pkv, pqf, e_tab)
    tg = jax.lax.bitcast_convert_type(t_g, jnp.float32)[:, 64:64 + K]
    dtf = (ts2 - tg).reshape(N2 * K, 1)

    S = 256
    grid = (N2 // S,)
    emb = pl.pallas_call(
        functools.partial(_attn_body, S, K, T, D_H),
        grid=grid,
        in_specs=[
            pl.BlockSpec((S, K, 128), lambda i: (i, 0, 0)),
            pl.BlockSpec((S, K, 128), lambda i: (i, 0, 0)),
            pl.BlockSpec((S * K, 1), lambda i: (i, 0)),
            pl.BlockSpec((S, 2 * D_H), lambda i: (i, 0)),
            pl.BlockSpec((1, T), lambda i: (0, 0)),
            pl.BlockSpec((1, T), lambda i: (0, 0)),
            pl.BlockSpec((T, 2 * D_H), lambda i: (0, 0)),
            pl.BlockSpec((D_E, 2 * D_H), lambda i: (0, 0)),
            pl.BlockSpec((T, D_H), lambda i: (0, 0)),
            pl.BlockSpec((D_H, D), lambda i: (0, 0)),
            pl.BlockSpec((1, D), lambda i: (0, 0)),
            pl.BlockSpec((D, D), lambda i: (0, 0)),
            pl.BlockSpec((1, D), lambda i: (0, 0)),
        ],
        out_specs=pl.BlockSpec((S, D), lambda i: (i, 0)),
        out_shape=jax.ShapeDtypeStruct((N2, D), jnp.float32),
    )(kv_g, e_g, dtf, qf_g, time_w.reshape(1, T), time_b.reshape(1, T),
      cwt, cwe, wqt, fc1_w[:D_H], fc1_b.reshape(1, D), fc2_w,
      fc2_b.reshape(1, D))

    return (emb[:B], emb[B:])


# speedup vs baseline: 8.5311x; 1.0102x over previous
import functools

import jax
import jax.numpy as jnp
from jax import lax
from jax.experimental import pallas as pl
from jax.experimental.pallas import tpu as pltpu
from jax.experimental.pallas import tpu_sc as plsc

_NC = 2
_NS = 16
_NW = _NC * _NS


def _proj_body(D_H, nf_ref, wkv_ref, wqf_ref, pkv_ref, pqf_ref):
    nf = nf_ref[...]
    kv = jax.lax.dot_general(
        nf, wkv_ref[...], (((1,), (0,)), ((), ())),
        preferred_element_type=jnp.float32)
    kw = jax.lax.bitcast_convert_type(
        kv[:, :D_H].astype(jnp.bfloat16), jnp.uint16).astype(jnp.uint32)
    vw = jax.lax.bitcast_convert_type(
        kv[:, D_H:].astype(jnp.bfloat16), jnp.uint16).astype(jnp.uint32)
    pkv_ref[...] = jax.lax.bitcast_convert_type(
        kw | (vw << 16), jnp.int32)
    pqf_ref[...] = jax.lax.dot_general(
        nf, wqf_ref[...], (((1,), (0,)), ((), ())),
        preferred_element_type=jnp.float32)


def _gather_body(C, SPW, K, nodes_hbm, misc_hbm, pkv_hbm, pqf_hbm, ef_hbm,
                 kv_out, e_out, t_out, qf_out,
                 idx_v, misc_v, qf_v, kv_v, e_v,
                 semL1a, semL1b, semL2a, semL2b, semSta, semStb):
    wid = lax.axis_index("s") * _NC + lax.axis_index("c")
    base = wid * SPW
    semL1 = (semL1a, semL1b)
    semL2 = (semL2a, semL2b)
    semSt = (semSta, semStb)
    NB = SPW // (2 * C)

    def store_descs(b, s0):
        out = [
            pltpu.make_async_copy(kv_v.at[b], kv_out.at[pl.ds(s0 + b * C, C)],
                                  semSt[b]),
            pltpu.make_async_copy(e_v.at[b], e_out.at[pl.ds(s0 + b * C, C)],
                                  semSt[b]),
        ]
        if b == 1:
            out.append(pltpu.make_async_copy(
                misc_v, t_out.at[pl.ds(s0, 2 * C)], semSt[b]))
            out.append(pltpu.make_async_copy(
                qf_v, qf_out.at[pl.ds(s0, 2 * C)], semSt[b]))
        return out

    def body(j, _):
        s0 = base + j * 2 * C
        @pl.when(j > 0)
        def _drain():
            for b in (0, 1):
                for d in store_descs(b, base + (j - 1) * 2 * C):
                    d.wait()
        pltpu.sync_copy(nodes_hbm.at[pl.ds(s0, 2 * C)], idx_v)
        d1 = pltpu.async_copy(misc_hbm.at[idx_v], misc_v, semL1[0])
        d2 = pltpu.async_copy(pqf_hbm.at[idx_v], qf_v, semL1[0])
        d1.wait()
        d2.wait()
        l2 = []
        for b in (0, 1):
            ds = []
            for i in range(C):
                ds.append(pltpu.async_copy(
                    pkv_hbm.at[misc_v.at[b * C + i, pl.ds(0, K)]],
                    kv_v.at[b, i], semL2[b]))
                ds.append(pltpu.async_copy(
                    ef_hbm.at[misc_v.at[b * C + i, pl.ds(32, K)]],
                    e_v.at[b, i], semL2[b]))
            l2.append(ds)
        for b in (0, 1):
            for d in l2[b]:
                d.wait()
            for d in store_descs(b, s0):
                d.start()
        return _

    lax.fori_loop(0, NB, body, 0)
    for b in (0, 1):
        for d in store_descs(b, base + (NB - 1) * 2 * C):
            d.wait()


def _fast_cos(x):
    n = jnp.round(x * 0.15915494309189535)
    r = x - n * 6.28125
    r = r - n * 1.9353071795864769e-03
    z = r * r
    p = jnp.float32(4.779477332387385e-14)
    for c in (-1.1470745597729725e-11, 2.08767569878681e-09,
              -2.755731922398589e-07, 2.48015873015873e-05,
              -1.388888888888889e-03, 4.1666666666666664e-02,
              -0.5, 1.0):
        p = p * z + c
    return p


def _attn_body(S, K, T, D_H, kv_ref, e_ref, dt_ref, qf_ref, w_ref,
               b_ref, cwt_ref, cwe_ref, wqt_ref, f1_ref, f1b_ref, f2_ref,
               f2b_ref, out_ref):
    b = b_ref[...]
    targ = dt_ref[...] * w_ref[...] + b
    tenc = _fast_cos(targ)
    proj = jax.lax.dot_general(
        tenc, cwt_ref[...], (((1,), (0,)), ((), ())),
        preferred_element_type=jnp.float32)
    D_E = cwe_ref.shape[0]
    proj += jax.lax.dot_general(
        e_ref[...][:, :, :D_E].reshape(S * K, D_E),
        cwe_ref[...], (((1,), (0,)), ((), ())),
        preferred_element_type=jnp.float32)
    kvp = jax.lax.bitcast_convert_type(kv_ref[...], jnp.uint32)
    kf = jax.lax.bitcast_convert_type(
        (kvp & jnp.uint32(0xFFFF)).astype(jnp.uint16),
        jnp.bfloat16).astype(jnp.float32)
    vf = jax.lax.bitcast_convert_type(
        (kvp >> 16).astype(jnp.uint16), jnp.bfloat16).astype(jnp.float32)
    kmat = kf + proj[:, :D_H].reshape(S, K, D_H)
    vmat = vf + proj[:, D_H:].reshape(S, K, D_H)
    qc = jax.lax.dot_general(
        jnp.cos(b), wqt_ref[...], (((1,), (0,)), ((), ())),
        preferred_element_type=jnp.float32)
    q = qf_ref[:, :D_H] + qc
    attn = jnp.sum(q[:, None, :] * kmat, axis=-1) * (D_H ** -0.5)
    m = jnp.max(attn, axis=-1, keepdims=True)
    p = jnp.exp(attn - m)
    a = p / jnp.sum(p, axis=-1, keepdims=True)
    agg = jnp.sum(a[:, :, None] * vmat, axis=1)
    h = jax.lax.dot_general(
        agg, f1_ref[...], (((1,), (0,)), ((), ())),
        preferred_element_type=jnp.float32)
    h = jnp.maximum(h + qf_ref[:, D_H:] + f1b_ref[0, :], 0.0)
    out = jax.lax.dot_general(
        h, f2_ref[...], (((1,), (0,)), ((), ())),
        preferred_element_type=jnp.float32)
    out_ref[...] = out + f2b_ref[0, :]


def kernel(source_nodes, destination_nodes, edge_times, edge_idxs,
           neighbor_nodes, neighbor_edge_idxs, neighbor_times,
           node_features, edge_features, time_w, time_b,
           Wq, Wk, Wv, fc1_w, fc1_b, fc2_w, fc2_b):
    del edge_idxs
    B = source_nodes.shape[0]
    N2 = 2 * B
    N, K = neighbor_nodes.shape
    D = node_features.shape[1]
    D_E = edge_features.shape[1]
    T = time_w.shape[0]
    D_H = Wq.shape[1]

    nodes = jnp.concatenate([source_nodes, destination_nodes]).astype(jnp.int32)
    ts2 = jnp.concatenate([edge_times, edge_times]).reshape(N2, 1)
    nbrn = neighbor_nodes.astype(jnp.int32)
    nbre = neighbor_edge_idxs.astype(jnp.int32)
    tbits = jax.lax.bitcast_convert_type(neighbor_times, jnp.int32)
    zpad = jnp.zeros((N, 32 - K), jnp.int32)
    misc_tab = jnp.concatenate(
        [nbrn, zpad, nbre, zpad, tbits, zpad,
         jnp.zeros((N, 32), jnp.int32)], axis=1)
    e_tab = jnp.pad(edge_features, ((0, 0), (0, 128 - D_E)))

    wkv = jnp.concatenate([Wk[:D], Wv[:D]], axis=1)
    wqf = jnp.concatenate([Wq[:D], fc1_w[D_H:]], axis=1)
    cwt = jnp.concatenate([Wk[D + D_E:], Wv[D + D_E:]], axis=1)
    cwe = jnp.concatenate([Wk[D:D + D_E], Wv[D:D + D_E]], axis=1)
    wqt = Wq[D:]

    pkv, pqf = pl.pallas_call(
        functools.partial(_proj_body, D_H),
        out_shape=(
            jax.ShapeDtypeStruct((N, D_H), jnp.int32),
            jax.ShapeDtypeStruct((N, 2 * D_H), jnp.float32),
        ),
    )(node_features, wkv, wqf)

    SPW = N2 // _NW
    C = 4
    mesh = plsc.VectorSubcoreMesh(core_axis_name="c", subcore_axis_name="s")
    gather = pl.kernel(
        functools.partial(_gather_body, C, SPW, K),
        out_type=(
            jax.ShapeDtypeStruct((N2, K, 128), jnp.int32),
            jax.ShapeDtypeStruct((N2, K, 128), jnp.float32),
            jax.ShapeDtypeStruct((N2, 128), jnp.int32),
            jax.ShapeDtypeStruct((N2, 2 * D_H), jnp.float32),
        ),
        mesh=mesh,
        scratch_types=[
            pltpu.VMEM((2 * C,), jnp.int32),
            pltpu.VMEM((2 * C, 128), jnp.int32),
            pltpu.VMEM((2 * C, 2 * D_H), jnp.float32),
            pltpu.VMEM((2, C, K, 128), jnp.int32),
            pltpu.VMEM((2, C, K, 128), jnp.float32),
            pltpu.SemaphoreType.DMA,
            pltpu.SemaphoreType.DMA,
            pltpu.SemaphoreType.DMA,
            pltpu.SemaphoreType.DMA,
            pltpu.SemaphoreType.DMA,
            pltpu.SemaphoreType.DMA,
        ],
    )
    kv_g, e_g, t_g, qf_g = gather(nodes, misc_tab, pkv, pqf, e_tab)
    tg = jax.lax.bitcast_convert_type(t_g, jnp.float32)[:, 64:64 + K]
    dtf = (ts2 - tg).reshape(N2 * K, 1)

    S = 256
    grid = (N2 // S,)
    emb = pl.pallas_call(
        functools.partial(_attn_body, S, K, T, D_H),
        grid=grid,
        in_specs=[
            pl.BlockSpec((S, K, 128), lambda i: (i, 0, 0)),
            pl.BlockSpec((S, K, 128), lambda i: (i, 0, 0)),
            pl.BlockSpec((S * K, 1), lambda i: (i, 0)),
            pl.BlockSpec((S, 2 * D_H), lambda i: (i, 0)),
            pl.BlockSpec((1, T), lambda i: (0, 0)),
            pl.BlockSpec((1, T), lambda i: (0, 0)),
            pl.BlockSpec((T, 2 * D_H), lambda i: (0, 0)),
            pl.BlockSpec((D_E, 2 * D_H), lambda i: (0, 0)),
            pl.BlockSpec((T, D_H), lambda i: (0, 0)),
            pl.BlockSpec((D_H, D), lambda i: (0, 0)),
            pl.BlockSpec((1, D), lambda i: (0, 0)),
            pl.BlockSpec((D, D), lambda i: (0, 0)),
            pl.BlockSpec((1, D), lambda i: (0, 0)),
        ],
        out_specs=pl.BlockSpec((S, D), lambda i: (i, 0)),
        out_shape=jax.ShapeDtypeStruct((N2, D), jnp.float32),
    )(kv_g, e_g, dtf, qf_g, time_w.reshape(1, T), time_b.reshape(1, T),
      cwt, cwe, wqt, fc1_w[:D_H], fc1_b.reshape(1, D), fc2_w,
      fc2_b.reshape(1, D))

    return (emb[:B], emb[B:])


# speedup vs baseline: 8.5595x; 1.0033x over previous
import functools

import jax
import jax.numpy as jnp
from jax import lax
from jax.experimental import pallas as pl
from jax.experimental.pallas import tpu as pltpu
from jax.experimental.pallas import tpu_sc as plsc

_NC = 2
_NS = 16
_NW = _NC * _NS


def _proj_body(D_H, nf_ref, wkv_ref, wqf_ref, pkv_ref, pqf_ref):
    nf = nf_ref[...]
    kv = jax.lax.dot_general(
        nf, wkv_ref[...], (((1,), (0,)), ((), ())),
        preferred_element_type=jnp.float32)
    kw = jax.lax.bitcast_convert_type(
        kv[:, :D_H].astype(jnp.bfloat16), jnp.uint16).astype(jnp.uint32)
    vw = jax.lax.bitcast_convert_type(
        kv[:, D_H:].astype(jnp.bfloat16), jnp.uint16).astype(jnp.uint32)
    pkv_ref[...] = jax.lax.bitcast_convert_type(
        kw | (vw << 16), jnp.int32)
    pqf_ref[...] = jax.lax.dot_general(
        nf, wqf_ref[...], (((1,), (0,)), ((), ())),
        preferred_element_type=jnp.float32)


def _gather_body(C, SPW, K, nodes_hbm, misc_hbm, pkv_hbm, pqf_hbm, ef_hbm,
                 kv_out, e_out, t_out, qf_out,
                 idx_v, misc_v, qf_v, kv_v, e_v,
                 semL1a, semL1b, semL2a, semL2b, semSta, semStb):
    wid = lax.axis_index("s") * _NC + lax.axis_index("c")
    base = wid * SPW
    semL1 = (semL1a, semL1b)
    semL2 = (semL2a, semL2b)
    semSt = (semSta, semStb)
    NB = SPW // (2 * C)

    def store_descs(b, s0):
        out = [
            pltpu.make_async_copy(kv_v.at[b], kv_out.at[pl.ds(s0 + b * C, C)],
                                  semSt[b]),
            pltpu.make_async_copy(e_v.at[b], e_out.at[pl.ds(s0 + b * C, C)],
                                  semSt[b]),
        ]
        if b == 1:
            out.append(pltpu.make_async_copy(
                misc_v, t_out.at[pl.ds(s0, 2 * C)], semSt[b]))
            out.append(pltpu.make_async_copy(
                qf_v, qf_out.at[pl.ds(s0, 2 * C)], semSt[b]))
        return out

    def body(j, _):
        s0 = base + j * 2 * C
        @pl.when(j > 0)
        def _drain():
            for b in (0, 1):
                for d in store_descs(b, base + (j - 1) * 2 * C):
                    d.wait()
        pltpu.sync_copy(nodes_hbm.at[pl.ds(s0, 2 * C)], idx_v)
        d1 = pltpu.async_copy(misc_hbm.at[idx_v], misc_v, semL1[0])
        d2 = pltpu.async_copy(pqf_hbm.at[idx_v], qf_v, semL1[0])
        d1.wait()
        d2.wait()
        l2 = []
        for b in (0, 1):
            ds = []
            for i in range(C):
                ds.append(pltpu.async_copy(
                    pkv_hbm.at[misc_v.at[b * C + i, pl.ds(0, K)]],
                    kv_v.at[b, i], semL2[b]))
                ds.append(pltpu.async_copy(
                    ef_hbm.at[misc_v.at[b * C + i, pl.ds(32, K)]],
                    e_v.at[b, i], semL2[b]))
            l2.append(ds)
        for b in (0, 1):
            for d in l2[b]:
                d.wait()
            for d in store_descs(b, s0):
                d.start()
        return _

    lax.fori_loop(0, NB, body, 0)
    for b in (0, 1):
        for d in store_descs(b, base + (NB - 1) * 2 * C):
            d.wait()


def _fast_cos(x):
    n = jnp.round(x * 0.15915494309189535)
    r = x - n * 6.28125
    r = r - n * 1.9353071795864769e-03
    z = r * r
    p = jnp.float32(4.779477332387385e-14)
    for c in (-1.1470745597729725e-11, 2.08767569878681e-09,
              -2.755731922398589e-07, 2.48015873015873e-05,
              -1.388888888888889e-03, 4.1666666666666664e-02,
              -0.5, 1.0):
        p = p * z + c
    return p


def _attn_body(S, K, T, D_H, kv_ref, e_ref, dt_ref, qf_ref, w_ref,
               b_ref, cwt_ref, cwe_ref, wqt_ref, f1_ref, f1b_ref, f2_ref,
               f2b_ref, out_ref):
    b = b_ref[...]
    targ = dt_ref[...] * w_ref[...] + b
    tenc = _fast_cos(targ)
    proj = jax.lax.dot_general(
        tenc, cwt_ref[...], (((1,), (0,)), ((), ())),
        preferred_element_type=jnp.float32)
    D_E = cwe_ref.shape[0]
    proj += jax.lax.dot_general(
        e_ref[...][:, :, :D_E].reshape(S * K, D_E),
        cwe_ref[...], (((1,), (0,)), ((), ())),
        preferred_element_type=jnp.float32)
    kvp = jax.lax.bitcast_convert_type(kv_ref[...], jnp.uint32)
    kf = jax.lax.bitcast_convert_type(
        (kvp & jnp.uint32(0xFFFF)).astype(jnp.uint16),
        jnp.bfloat16).astype(jnp.float32)
    vf = jax.lax.bitcast_convert_type(
        (kvp >> 16).astype(jnp.uint16), jnp.bfloat16).astype(jnp.float32)
    kmat = kf + proj[:, :D_H].reshape(S, K, D_H)
    vmat = vf + proj[:, D_H:].reshape(S, K, D_H)
    qc = jax.lax.dot_general(
        jnp.cos(b), wqt_ref[...], (((1,), (0,)), ((), ())),
        preferred_element_type=jnp.float32)
    q = qf_ref[:, :D_H] + qc
    attn = jnp.sum(q[:, None, :] * kmat, axis=-1) * (D_H ** -0.5)
    m = jnp.max(attn, axis=-1, keepdims=True)
    p = jnp.exp(attn - m)
    a = p / jnp.sum(p, axis=-1, keepdims=True)
    agg = jnp.sum(a[:, :, None] * vmat, axis=1)
    h = jax.lax.dot_general(
        agg, f1_ref[...], (((1,), (0,)), ((), ())),
        preferred_element_type=jnp.float32)
    h = jnp.maximum(h + qf_ref[:, D_H:] + f1b_ref[0, :], 0.0)
    out = jax.lax.dot_general(
        h, f2_ref[...], (((1,), (0,)), ((), ())),
        preferred_element_type=jnp.float32)
    out_ref[...] = out + f2b_ref[0, :]


def kernel(source_nodes, destination_nodes, edge_times, edge_idxs,
           neighbor_nodes, neighbor_edge_idxs, neighbor_times,
           node_features, edge_features, time_w, time_b,
           Wq, Wk, Wv, fc1_w, fc1_b, fc2_w, fc2_b):
    del edge_idxs
    B = source_nodes.shape[0]
    N2 = 2 * B
    N, K = neighbor_nodes.shape
    D = node_features.shape[1]
    D_E = edge_features.shape[1]
    T = time_w.shape[0]
    D_H = Wq.shape[1]

    nodes = jnp.concatenate([source_nodes, destination_nodes]).astype(jnp.int32)
    ts2 = jnp.concatenate([edge_times, edge_times]).reshape(N2, 1)
    nbrn = neighbor_nodes.astype(jnp.int32)
    nbre = neighbor_edge_idxs.astype(jnp.int32)
    tbits = jax.lax.bitcast_convert_type(neighbor_times, jnp.int32)
    zpad = jnp.zeros((N, 32 - K), jnp.int32)
    misc_tab = jnp.concatenate(
        [nbrn, zpad, nbre, zpad, tbits, zpad,
         jnp.zeros((N, 32), jnp.int32)], axis=1)
    e_tab = jnp.pad(edge_features, ((0, 0), (0, 128 - D_E)))

    wkv = jnp.concatenate([Wk[:D], Wv[:D]], axis=1)
    wqf = jnp.concatenate([Wq[:D], fc1_w[D_H:]], axis=1)
    cwt = jnp.concatenate([Wk[D + D_E:], Wv[D + D_E:]], axis=1)
    cwe = jnp.concatenate([Wk[D:D + D_E], Wv[D:D + D_E]], axis=1)
    wqt = Wq[D:]

    pkv, pqf = pl.pallas_call(
        functools.partial(_proj_body, D_H),
        out_shape=(
            jax.ShapeDtypeStruct((N, D_H), jnp.int32),
            jax.ShapeDtypeStruct((N, 2 * D_H), jnp.float32),
        ),
    )(node_features, wkv, wqf)

    SPW = B // _NW
    C = 4
    mesh = plsc.VectorSubcoreMesh(core_axis_name="c", subcore_axis_name="s")
    gather = pl.kernel(
        functools.partial(_gather_body, C, SPW, K),
        out_type=(
            jax.ShapeDtypeStruct((B, K, 128), jnp.int32),
            jax.ShapeDtypeStruct((B, K, 128), jnp.float32),
            jax.ShapeDtypeStruct((B, 128), jnp.int32),
            jax.ShapeDtypeStruct((B, 2 * D_H), jnp.float32),
        ),
        mesh=mesh,
        scratch_types=[
            pltpu.VMEM((2 * C,), jnp.int32),
            pltpu.VMEM((2 * C, 128), jnp.int32),
            pltpu.VMEM((2 * C, 2 * D_H), jnp.float32),
            pltpu.VMEM((2, C, K, 128), jnp.int32),
            pltpu.VMEM((2, C, K, 128), jnp.float32),
            pltpu.SemaphoreType.DMA,
            pltpu.SemaphoreType.DMA,
            pltpu.SemaphoreType.DMA,
            pltpu.SemaphoreType.DMA,
            pltpu.SemaphoreType.DMA,
            pltpu.SemaphoreType.DMA,
        ],
    )
    S = 256
    attn = pl.pallas_call(
        functools.partial(_attn_body, S, K, T, D_H),
        grid=(B // S,),
        in_specs=[
            pl.BlockSpec((S, K, 128), lambda i: (i, 0, 0)),
            pl.BlockSpec((S, K, 128), lambda i: (i, 0, 0)),
            pl.BlockSpec((S * K, 1), lambda i: (i, 0)),
            pl.BlockSpec((S, 2 * D_H), lambda i: (i, 0)),
            pl.BlockSpec((1, T), lambda i: (0, 0)),
            pl.BlockSpec((1, T), lambda i: (0, 0)),
            pl.BlockSpec((T, 2 * D_H), lambda i: (0, 0)),
            pl.BlockSpec((D_E, 2 * D_H), lambda i: (0, 0)),
            pl.BlockSpec((T, D_H), lambda i: (0, 0)),
            pl.BlockSpec((D_H, D), lambda i: (0, 0)),
            pl.BlockSpec((1, D), lambda i: (0, 0)),
            pl.BlockSpec((D, D), lambda i: (0, 0)),
            pl.BlockSpec((1, D), lambda i: (0, 0)),
        ],
        out_specs=pl.BlockSpec((S, D), lambda i: (i, 0)),
        out_shape=jax.ShapeDtypeStruct((B, D), jnp.float32),
    )

    ts1 = edge_times.reshape(B, 1)
    gathered = [gather(n, misc_tab, pkv, pqf, e_tab)
                for n in (nodes[:B], nodes[B:])]
    embs = []
    for kv_g, e_g, t_g, qf_g in gathered:
        tg = jax.lax.bitcast_convert_type(t_g, jnp.float32)[:, 64:64 + K]
        dtf = (ts1 - tg).reshape(B * K, 1)
        embs.append(attn(
            kv_g, e_g, dtf, qf_g, time_w.reshape(1, T), time_b.reshape(1, T),
            cwt, cwe, wqt, fc1_w[:D_H], fc1_b.reshape(1, D), fc2_w,
            fc2_b.reshape(1, D)))

    return (embs[0], embs[1])


# speedup vs baseline: 8.5749x; 1.0018x over previous
import functools

import jax
import jax.numpy as jnp
from jax import lax
from jax.experimental import pallas as pl
from jax.experimental.pallas import tpu as pltpu
from jax.experimental.pallas import tpu_sc as plsc

_NC = 2
_NS = 16
_NW = _NC * _NS


def _proj_body(D_H, nf_ref, wkv_ref, wqf_ref, pkv_ref, pqf_ref):
    nf = nf_ref[...]
    kv = jax.lax.dot_general(
        nf, wkv_ref[...], (((1,), (0,)), ((), ())),
        preferred_element_type=jnp.float32)
    kw = jax.lax.bitcast_convert_type(
        kv[:, :D_H].astype(jnp.bfloat16), jnp.uint16).astype(jnp.uint32)
    vw = jax.lax.bitcast_convert_type(
        kv[:, D_H:].astype(jnp.bfloat16), jnp.uint16).astype(jnp.uint32)
    pkv_ref[...] = jax.lax.bitcast_convert_type(
        kw | (vw << 16), jnp.int32)
    pqf_ref[...] = jax.lax.dot_general(
        nf, wqf_ref[...], (((1,), (0,)), ((), ())),
        preferred_element_type=jnp.float32)


def _gather_body(C, SPW, K, nodes_hbm, misc_hbm, pkv_hbm, pqf_hbm, ef_hbm,
                 kv_out, e_out, t_out, qf_out,
                 idx_v, misc_v, qf_v, kv_v, e_v,
                 semL1a, semL1b, semL2a, semL2b, semSta, semStb):
    wid = lax.axis_index("s") * _NC + lax.axis_index("c")
    base = wid * SPW
    semL1 = (semL1a, semL1b)
    semL2 = (semL2a, semL2b)
    semSt = (semSta, semStb)
    NB = SPW // (2 * C)

    def store_descs(b, s0):
        out = [
            pltpu.make_async_copy(kv_v.at[b], kv_out.at[pl.ds(s0 + b * C, C)],
                                  semSt[b]),
            pltpu.make_async_copy(e_v.at[b], e_out.at[pl.ds(s0 + b * C, C)],
                                  semSt[b]),
        ]
        if b == 1:
            out.append(pltpu.make_async_copy(
                misc_v, t_out.at[pl.ds(s0, 2 * C)], semSt[b]))
            out.append(pltpu.make_async_copy(
                qf_v, qf_out.at[pl.ds(s0, 2 * C)], semSt[b]))
        return out

    def body(j, _):
        s0 = base + j * 2 * C
        @pl.when(j > 0)
        def _drain():
            for b in (0, 1):
                for d in store_descs(b, base + (j - 1) * 2 * C):
                    d.wait()
        pltpu.sync_copy(nodes_hbm.at[pl.ds(s0, 2 * C)], idx_v)
        d1 = pltpu.async_copy(misc_hbm.at[idx_v], misc_v, semL1[0])
        d2 = pltpu.async_copy(pqf_hbm.at[idx_v], qf_v, semL1[0])
        d1.wait()
        d2.wait()
        l2 = []
        for b in (0, 1):
            ds = []
            for i in range(C):
                ds.append(pltpu.async_copy(
                    pkv_hbm.at[misc_v.at[b * C + i, pl.ds(0, K)]],
                    kv_v.at[b, i], semL2[b]))
                ds.append(pltpu.async_copy(
                    ef_hbm.at[misc_v.at[b * C + i, pl.ds(32, K)]],
                    e_v.at[b, i], semL2[b]))
            l2.append(ds)
        for b in (0, 1):
            for d in l2[b]:
                d.wait()
            for d in store_descs(b, s0):
                d.start()
        return _

    lax.fori_loop(0, NB, body, 0)
    for b in (0, 1):
        for d in store_descs(b, base + (NB - 1) * 2 * C):
            d.wait()


def _fast_cos(x):
    n = jnp.round(x * 0.15915494309189535)
    r = x - n * 6.28125
    r = r - n * 1.9353071795864769e-03
    z = r * r
    p = jnp.float32(4.779477332387385e-14)
    for c in (-1.1470745597729725e-11, 2.08767569878681e-09,
              -2.755731922398589e-07, 2.48015873015873e-05,
              -1.388888888888889e-03, 4.1666666666666664e-02,
              -0.5, 1.0):
        p = p * z + c
    return p


def _attn_body(S, K, T, D_H, kv_ref, e_ref, dt_ref, qf_ref, w_ref,
               b_ref, cwt_ref, cwe_ref, wqt_ref, f1_ref, f1b_ref, f2_ref,
               f2b_ref, out_ref):
    b = b_ref[...]
    targ = dt_ref[...] * w_ref[...] + b
    tenc = _fast_cos(targ)
    proj = jax.lax.dot_general(
        tenc, cwt_ref[...], (((1,), (0,)), ((), ())),
        preferred_element_type=jnp.float32)
    proj += jax.lax.dot_general(
        e_ref[...].reshape(S * K, 128),
        cwe_ref[...], (((1,), (0,)), ((), ())),
        preferred_element_type=jnp.float32)
    kvp = jax.lax.bitcast_convert_type(kv_ref[...], jnp.uint32)
    kf = jax.lax.bitcast_convert_type(
        (kvp & jnp.uint32(0xFFFF)).astype(jnp.uint16),
        jnp.bfloat16).astype(jnp.float32)
    vf = jax.lax.bitcast_convert_type(
        (kvp >> 16).astype(jnp.uint16), jnp.bfloat16).astype(jnp.float32)
    kmat = kf + proj[:, :D_H].reshape(S, K, D_H)
    vmat = vf + proj[:, D_H:].reshape(S, K, D_H)
    qc = jax.lax.dot_general(
        jnp.cos(b), wqt_ref[...], (((1,), (0,)), ((), ())),
        preferred_element_type=jnp.float32)
    q = qf_ref[:, :D_H] + qc
    attn = jnp.sum(q[:, None, :] * kmat, axis=-1) * (D_H ** -0.5)
    m = jnp.max(attn, axis=-1, keepdims=True)
    p = jnp.exp2((attn - m) * 1.4426950408889634)
    a = p / jnp.sum(p, axis=-1, keepdims=True)
    agg = jnp.sum(a[:, :, None] * vmat, axis=1)
    h = jax.lax.dot_general(
        agg, f1_ref[...], (((1,), (0,)), ((), ())),
        preferred_element_type=jnp.float32)
    h = jnp.maximum(h + qf_ref[:, D_H:] + f1b_ref[0, :], 0.0)
    out = jax.lax.dot_general(
        h, f2_ref[...], (((1,), (0,)), ((), ())),
        preferred_element_type=jnp.float32)
    out_ref[...] = out + f2b_ref[0, :]


def kernel(source_nodes, destination_nodes, edge_times, edge_idxs,
           neighbor_nodes, neighbor_edge_idxs, neighbor_times,
           node_features, edge_features, time_w, time_b,
           Wq, Wk, Wv, fc1_w, fc1_b, fc2_w, fc2_b):
    del edge_idxs
    B = source_nodes.shape[0]
    N2 = 2 * B
    N, K = neighbor_nodes.shape
    D = node_features.shape[1]
    D_E = edge_features.shape[1]
    T = time_w.shape[0]
    D_H = Wq.shape[1]

    nodes = jnp.concatenate([source_nodes, destination_nodes]).astype(jnp.int32)
    ts2 = jnp.concatenate([edge_times, edge_times]).reshape(N2, 1)
    nbrn = neighbor_nodes.astype(jnp.int32)
    nbre = neighbor_edge_idxs.astype(jnp.int32)
    tbits = jax.lax.bitcast_convert_type(neighbor_times, jnp.int32)
    zpad = jnp.zeros((N, 32 - K), jnp.int32)
    misc_tab = jnp.concatenate(
        [nbrn, zpad, nbre, zpad, tbits, zpad,
         jnp.zeros((N, 32), jnp.int32)], axis=1)
    e_tab = jnp.pad(edge_features, ((0, 0), (0, 128 - D_E)))

    wkv = jnp.concatenate([Wk[:D], Wv[:D]], axis=1)
    wqf = jnp.concatenate([Wq[:D], fc1_w[D_H:]], axis=1)
    cwt = jnp.concatenate([Wk[D + D_E:], Wv[D + D_E:]], axis=1)
    cwe = jnp.concatenate([
        jnp.concatenate([Wk[D:D + D_E], Wv[D:D + D_E]], axis=1),
        jnp.zeros((128 - D_E, 2 * D_H), jnp.float32)], axis=0)
    wqt = Wq[D:]

    pkv, pqf = pl.pallas_call(
        functools.partial(_proj_body, D_H),
        out_shape=(
            jax.ShapeDtypeStruct((N, D_H), jnp.int32),
            jax.ShapeDtypeStruct((N, 2 * D_H), jnp.float32),
        ),
    )(node_features, wkv, wqf)

    SPW = B // _NW
    C = 4
    mesh = plsc.VectorSubcoreMesh(core_axis_name="c", subcore_axis_name="s")
    gather = pl.kernel(
        functools.partial(_gather_body, C, SPW, K),
        out_type=(
            jax.ShapeDtypeStruct((B, K, 128), jnp.int32),
            jax.ShapeDtypeStruct((B, K, 128), jnp.float32),
            jax.ShapeDtypeStruct((B, 128), jnp.int32),
            jax.ShapeDtypeStruct((B, 2 * D_H), jnp.float32),
        ),
        mesh=mesh,
        scratch_types=[
            pltpu.VMEM((2 * C,), jnp.int32),
            pltpu.VMEM((2 * C, 128), jnp.int32),
            pltpu.VMEM((2 * C, 2 * D_H), jnp.float32),
            pltpu.VMEM((2, C, K, 128), jnp.int32),
            pltpu.VMEM((2, C, K, 128), jnp.float32),
            pltpu.SemaphoreType.DMA,
            pltpu.SemaphoreType.DMA,
            pltpu.SemaphoreType.DMA,
            pltpu.SemaphoreType.DMA,
            pltpu.SemaphoreType.DMA,
            pltpu.SemaphoreType.DMA,
        ],
    )
    S = 256
    attn = pl.pallas_call(
        functools.partial(_attn_body, S, K, T, D_H),
        grid=(B // S,),
        in_specs=[
            pl.BlockSpec((S, K, 128), lambda i: (i, 0, 0)),
            pl.BlockSpec((S, K, 128), lambda i: (i, 0, 0)),
            pl.BlockSpec((S * K, 1), lambda i: (i, 0)),
            pl.BlockSpec((S, 2 * D_H), lambda i: (i, 0)),
            pl.BlockSpec((1, T), lambda i: (0, 0)),
            pl.BlockSpec((1, T), lambda i: (0, 0)),
            pl.BlockSpec((T, 2 * D_H), lambda i: (0, 0)),
            pl.BlockSpec((128, 2 * D_H), lambda i: (0, 0)),
            pl.BlockSpec((T, D_H), lambda i: (0, 0)),
            pl.BlockSpec((D_H, D), lambda i: (0, 0)),
            pl.BlockSpec((1, D), lambda i: (0, 0)),
            pl.BlockSpec((D, D), lambda i: (0, 0)),
            pl.BlockSpec((1, D), lambda i: (0, 0)),
        ],
        out_specs=pl.BlockSpec((S, D), lambda i: (i, 0)),
        out_shape=jax.ShapeDtypeStruct((B, D), jnp.float32),
    )

    ts1 = edge_times.reshape(B, 1)
    gathered = [gather(n, misc_tab, pkv, pqf, e_tab)
                for n in (nodes[:B], nodes[B:])]
    embs = []
    for kv_g, e_g, t_g, qf_g in gathered:
        tg = jax.lax.bitcast_convert_type(t_g, jnp.float32)[:, 64:64 + K]
        dtf = (ts1 - tg).reshape(B * K, 1)
        embs.append(attn(
            kv_g, e_g, dtf, qf_g, time_w.reshape(1, T), time_b.reshape(1, T),
            cwt, cwe, wqt, fc1_w[:D_H], fc1_b.reshape(1, D), fc2_w,
            fc2_b.reshape(1, D)))

    return (embs[0], embs[1])


# speedup vs baseline: 8.5859x; 1.0013x over previous
import functools

import jax
import jax.numpy as jnp
from jax import lax
from jax.experimental import pallas as pl
from jax.experimental.pallas import tpu as pltpu
from jax.experimental.pallas import tpu_sc as plsc

_NC = 2
_NS = 16
_NW = _NC * _NS


def _proj_body(D_H, nf_ref, wkv_ref, wqf_ref, pkv_ref, pqf_ref):
    nf = nf_ref[...]
    kv = jax.lax.dot_general(
        nf, wkv_ref[...], (((1,), (0,)), ((), ())),
        preferred_element_type=jnp.float32)
    kw = jax.lax.bitcast_convert_type(
        kv[:, :D_H].astype(jnp.bfloat16), jnp.uint16).astype(jnp.uint32)
    vw = jax.lax.bitcast_convert_type(
        kv[:, D_H:].astype(jnp.bfloat16), jnp.uint16).astype(jnp.uint32)
    pkv_ref[...] = jax.lax.bitcast_convert_type(
        kw | (vw << 16), jnp.int32)
    pqf_ref[...] = jax.lax.dot_general(
        nf, wqf_ref[...], (((1,), (0,)), ((), ())),
        preferred_element_type=jnp.float32)


def _gather_body(C, SPW, K, nodes_hbm, misc_hbm, pkv_hbm, pqf_hbm, ef_hbm,
                 kv_out, e_out, t_out, qf_out,
                 idx_v, misc_v, qf_v, kv_v, e_v,
                 semL1a, semL1b, semL2a, semL2b, semSta, semStb):
    wid = lax.axis_index("s") * _NC + lax.axis_index("c")
    base = wid * SPW
    semL1 = (semL1a, semL1b)
    semL2 = (semL2a, semL2b)
    semSt = (semSta, semStb)
    NB = SPW // (2 * C)

    def store_descs(b, s0):
        out = [
            pltpu.make_async_copy(kv_v.at[b], kv_out.at[pl.ds(s0 + b * C, C)],
                                  semSt[b]),
            pltpu.make_async_copy(e_v.at[b], e_out.at[pl.ds(s0 + b * C, C)],
                                  semSt[b]),
        ]
        if b == 1:
            out.append(pltpu.make_async_copy(
                misc_v, t_out.at[pl.ds(s0, 2 * C)], semSt[b]))
            out.append(pltpu.make_async_copy(
                qf_v, qf_out.at[pl.ds(s0, 2 * C)], semSt[b]))
        return out

    def body(j, _):
        s0 = base + j * 2 * C
        @pl.when(j > 0)
        def _drain():
            for b in (0, 1):
                for d in store_descs(b, base + (j - 1) * 2 * C):
                    d.wait()
        pltpu.sync_copy(nodes_hbm.at[pl.ds(s0, 2 * C)], idx_v)
        d1 = pltpu.async_copy(misc_hbm.at[idx_v], misc_v, semL1[0])
        d2 = pltpu.async_copy(pqf_hbm.at[idx_v], qf_v, semL1[0])
        d1.wait()
        d2.wait()
        l2 = []
        for b in (0, 1):
            ds = []
            for i in range(C):
                ds.append(pltpu.async_copy(
                    pkv_hbm.at[misc_v.at[b * C + i, pl.ds(0, K)]],
                    kv_v.at[b, i], semL2[b]))
                ds.append(pltpu.async_copy(
                    ef_hbm.at[misc_v.at[b * C + i, pl.ds(32, K)]],
                    e_v.at[b, i], semL2[b]))
            l2.append(ds)
        for b in (0, 1):
            for d in l2[b]:
                d.wait()
            for d in store_descs(b, s0):
                d.start()
        return _

    lax.fori_loop(0, NB, body, 0)
    for b in (0, 1):
        for d in store_descs(b, base + (NB - 1) * 2 * C):
            d.wait()


def _fast_cos(x):
    n = jnp.round(x * 0.15915494309189535)
    r = x - n * 6.28125
    r = r - n * 1.9353071795864769e-03
    z = r * r
    p = jnp.float32(4.779477332387385e-14)
    for c in (-1.1470745597729725e-11, 2.08767569878681e-09,
              -2.755731922398589e-07, 2.48015873015873e-05,
              -1.388888888888889e-03, 4.1666666666666664e-02,
              -0.5, 1.0):
        p = p * z + c
    return p


def _attn_body(S, K, T, D_H, kv_ref, e_ref, dt_ref, qf_ref, w_ref,
               b_ref, cwt_ref, cwe_ref, wqt_ref, f1_ref, f1b_ref, f2_ref,
               f2b_ref, out_ref):
    b = b_ref[...]
    targ = dt_ref[...] * w_ref[...] + b
    tenc = _fast_cos(targ)
    proj = jax.lax.dot_general(
        tenc, cwt_ref[...], (((1,), (0,)), ((), ())),
        preferred_element_type=jnp.float32)
    proj += jax.lax.dot_general(
        e_ref[...].reshape(S * K, 128),
        cwe_ref[...], (((1,), (0,)), ((), ())),
        preferred_element_type=jnp.float32)
    kvp = jax.lax.bitcast_convert_type(kv_ref[...], jnp.uint32)
    kf = jax.lax.bitcast_convert_type(
        (kvp & jnp.uint32(0xFFFF)).astype(jnp.uint16),
        jnp.bfloat16).astype(jnp.float32)
    vf = jax.lax.bitcast_convert_type(
        (kvp >> 16).astype(jnp.uint16), jnp.bfloat16).astype(jnp.float32)
    kmat = kf + proj[:, :D_H].reshape(S, K, D_H)
    vmat = vf + proj[:, D_H:].reshape(S, K, D_H)
    qc = jax.lax.dot_general(
        jnp.cos(b), wqt_ref[...], (((1,), (0,)), ((), ())),
        preferred_element_type=jnp.float32)
    q = qf_ref[:, :D_H] + qc
    attn = jnp.sum(q[:, None, :] * kmat, axis=-1) * (D_H ** -0.5)
    m = jnp.max(attn, axis=-1, keepdims=True)
    p = jnp.exp2((attn - m) * 1.4426950408889634)
    a = p / jnp.sum(p, axis=-1, keepdims=True)
    agg = jnp.sum(a[:, :, None] * vmat, axis=1)
    h = jax.lax.dot_general(
        agg, f1_ref[...], (((1,), (0,)), ((), ())),
        preferred_element_type=jnp.float32)
    h = jnp.maximum(h + qf_ref[:, D_H:] + f1b_ref[0, :], 0.0)
    out = jax.lax.dot_general(
        h, f2_ref[...], (((1,), (0,)), ((), ())),
        preferred_element_type=jnp.float32)
    out_ref[...] = out + f2b_ref[0, :]


def kernel(source_nodes, destination_nodes, edge_times, edge_idxs,
           neighbor_nodes, neighbor_edge_idxs, neighbor_times,
           node_features, edge_features, time_w, time_b,
           Wq, Wk, Wv, fc1_w, fc1_b, fc2_w, fc2_b):
    del edge_idxs
    B = source_nodes.shape[0]
    N, K = neighbor_nodes.shape
    D = node_features.shape[1]
    D_E = edge_features.shape[1]
    T = time_w.shape[0]
    D_H = Wq.shape[1]

    nodes = jnp.concatenate([source_nodes, destination_nodes]).astype(jnp.int32)
    nbrn = neighbor_nodes.astype(jnp.int32)
    nbre = neighbor_edge_idxs.astype(jnp.int32)
    tbits = jax.lax.bitcast_convert_type(neighbor_times, jnp.int32)
    zpad = jnp.zeros((N, 32 - K), jnp.int32)
    misc_tab = jnp.concatenate(
        [nbrn, zpad, nbre, zpad, tbits, zpad,
         jnp.zeros((N, 32), jnp.int32)], axis=1)
    e_tab = jnp.pad(edge_features, ((0, 0), (0, 128 - D_E)))

    wkv = jnp.concatenate([Wk[:D], Wv[:D]], axis=1)
    wqf = jnp.concatenate([Wq[:D], fc1_w[D_H:]], axis=1)
    cwt = jnp.concatenate([Wk[D + D_E:], Wv[D + D_E:]], axis=1)
    cwe = jnp.concatenate([
        jnp.concatenate([Wk[D:D + D_E], Wv[D:D + D_E]], axis=1),
        jnp.zeros((128 - D_E, 2 * D_H), jnp.float32)], axis=0)
    wqt = Wq[D:]

    pkv, pqf = pl.pallas_call(
        functools.partial(_proj_body, D_H),
        out_shape=(
            jax.ShapeDtypeStruct((N, D_H), jnp.int32),
            jax.ShapeDtypeStruct((N, 2 * D_H), jnp.float32),
        ),
    )(node_features, wkv, wqf)

    SPW = B // _NW
    C = 4
    mesh = plsc.VectorSubcoreMesh(core_axis_name="c", subcore_axis_name="s")
    gather = pl.kernel(
        functools.partial(_gather_body, C, SPW, K),
        out_type=(
            jax.ShapeDtypeStruct((B, K, 128), jnp.int32),
            jax.ShapeDtypeStruct((B, K, 128), jnp.float32),
            jax.ShapeDtypeStruct((B, 128), jnp.int32),
            jax.ShapeDtypeStruct((B, 2 * D_H), jnp.float32),
        ),
        mesh=mesh,
        scratch_types=[
            pltpu.VMEM((2 * C,), jnp.int32),
            pltpu.VMEM((2 * C, 128), jnp.int32),
            pltpu.VMEM((2 * C, 2 * D_H), jnp.float32),
            pltpu.VMEM((2, C, K, 128), jnp.int32),
            pltpu.VMEM((2, C, K, 128), jnp.float32),
            pltpu.SemaphoreType.DMA,
            pltpu.SemaphoreType.DMA,
            pltpu.SemaphoreType.DMA,
            pltpu.SemaphoreType.DMA,
            pltpu.SemaphoreType.DMA,
            pltpu.SemaphoreType.DMA,
        ],
    )
    S = 256
    attn = pl.pallas_call(
        functools.partial(_attn_body, S, K, T, D_H),
        grid=(B // S,),
        in_specs=[
            pl.BlockSpec((S, K, 128), lambda i: (i, 0, 0)),
            pl.BlockSpec((S, K, 128), lambda i: (i, 0, 0)),
            pl.BlockSpec((S * K, 1), lambda i: (i, 0)),
            pl.BlockSpec((S, 2 * D_H), lambda i: (i, 0)),
            pl.BlockSpec((1, T), lambda i: (0, 0)),
            pl.BlockSpec((1, T), lambda i: (0, 0)),
            pl.BlockSpec((T, 2 * D_H), lambda i: (0, 0)),
            pl.BlockSpec((128, 2 * D_H), lambda i: (0, 0)),
            pl.BlockSpec((T, D_H), lambda i: (0, 0)),
            pl.BlockSpec((D_H, D), lambda i: (0, 0)),
            pl.BlockSpec((1, D), lambda i: (0, 0)),
            pl.BlockSpec((D, D), lambda i: (0, 0)),
            pl.BlockSpec((1, D), lambda i: (0, 0)),
        ],
        out_specs=pl.BlockSpec((S, D), lambda i: (i, 0)),
        out_shape=jax.ShapeDtypeStruct((B, D), jnp.float32),
    )

    ts1 = edge_times.reshape(B, 1)
    gathered = [gather(n, misc_tab, pkv, pqf, e_tab)
                for n in (nodes[:B], nodes[B:])]
    embs = []
    for kv_g, e_g, t_g, qf_g in gathered:
        tg = jax.lax.bitcast_convert_type(t_g, jnp.float32)[:, 64:64 + K]
        dtf = (ts1 - tg).reshape(B * K, 1)
        embs.append(attn(
            kv_g, e_g, dtf, qf_g, time_w.reshape(1, T), time_b.reshape(1, T),
            cwt, cwe, wqt, fc1_w[:D_H], fc1_b.reshape(1, D), fc2_w,
            fc2_b.reshape(1, D)))

    return (embs[0], embs[1])
